# Initial kernel scaffold; baseline (speedup 1.0000x reference)
import jax
import jax.numpy as jnp
from jax.experimental import pallas as pl


def kernel(inputs, W):
    raise NotImplementedError("write your pallas kernel here")


# speedup vs baseline: 2.7022x; 2.7022x over previous
import jax
import jax.numpy as jnp
from jax.experimental import pallas as pl
from jax.experimental.pallas import tpu as pltpu

_NE = 1024
_D = 64
_BLK = 2048
_NTOK = 32768
_NELEM = 32 * 64 * 32 * 32


def _vq_body(x_ref, w_ref, q_ref, loss_ref, acc_ref):
    i = pl.program_id(0)
    n = pl.num_programs(0)
    x = x_ref[...]
    w = w_ref[...]
    xsq = jnp.sum(x * x, axis=1, keepdims=True)
    wsq = jnp.sum(w * w, axis=1, keepdims=True)
    m = jax.lax.dot_general(x, w, (((1,), (1,)), ((), ())),
                            preferred_element_type=jnp.float32)
    dist = (xsq + wsq.reshape(1, _NE)) - 2.0 * m + 1e-8
    dmin = jnp.min(dist, axis=1, keepdims=True)
    ids = jax.lax.broadcasted_iota(jnp.int32, dist.shape, 1)
    idx = jnp.min(jnp.where(dist == dmin, ids, _NE), axis=1, keepdims=True)
    oh = (ids == idx).astype(jnp.float32)
    q = jax.lax.dot_general(oh, w, (((1,), (0,)), ((), ())),
                            preferred_element_type=jnp.float32)
    q_ref[...] = x + (q - x)
    diff = q - x
    part = jnp.sum(jnp.sum(diff * diff, axis=1, keepdims=True),
                   axis=0, keepdims=True)

    @pl.when(i == 0)
    def _init():
        acc_ref[0, 0] = 0.0

    acc_ref[0, 0] += part[0, 0]

    @pl.when(i == n - 1)
    def _fini():
        mean = acc_ref[0, 0] * (1.0 / _NELEM)
        loss_ref[0, 0] = mean + 0.25 * mean


def kernel(inputs, W):
    shp = inputs.shape
    flat = jnp.transpose(inputs, (0, 2, 3, 1)).reshape(-1, _D)
    grid = _NTOK // _BLK
    q, loss = pl.pallas_call(
        _vq_body,
        grid=(grid,),
        in_specs=[
            pl.BlockSpec((_BLK, _D), lambda i: (i, 0)),
            pl.BlockSpec((_NE, _D), lambda i: (0, 0)),
        ],
        out_specs=[
            pl.BlockSpec((_BLK, _D), lambda i: (i, 0)),
            pl.BlockSpec(memory_space=pltpu.SMEM),
        ],
        out_shape=[
            jax.ShapeDtypeStruct((_NTOK, _D), jnp.float32),
            jax.ShapeDtypeStruct((1, 1), jnp.float32),
        ],
        scratch_shapes=[pltpu.SMEM((1, 1), jnp.float32)],
    )(flat, W)
    qst = jnp.transpose(q.reshape(shp[0], shp[2], shp[3], _D), (0, 3, 1, 2))
    return qst, loss[0, 0]
